# Initial kernel scaffold; baseline (speedup 1.0000x reference)
import jax
import jax.numpy as jnp
from jax.experimental import pallas as pl


def kernel(output, target):
    raise NotImplementedError("write your pallas kernel here")


# speedup vs baseline: 37.5449x; 37.5449x over previous
import functools

import jax
import jax.numpy as jnp
from jax import lax
from jax.experimental import pallas as pl
from jax.experimental.pallas import tpu as pltpu
from jax.experimental.pallas import tpu_sc as plsc

NBINS = 256
BATCH, CHANS, H, W = 8, 3, 512, 512
CHAN_ELEMS = H * W
N_CHAN = BATCH * CHANS
N_TOT = N_CHAN * CHAN_ELEMS
NC, NS, L = 2, 16, 16
CH_PER_CORE = N_CHAN // NC
CORE_ELEMS = CH_PER_CORE * CHAN_ELEMS
SUB_ELEMS = CORE_ELEMS // NS
BLK = 16384
NBLK = SUB_ELEMS // BLK
VPB = BLK // L
UNROLL = 8
HBINS = CH_PER_CORE * NBINS
SLICE = HBINS // NS
SCALE = 1.0 / float(N_CHAN * NBINS * CHAN_ELEMS)

_mesh = plsc.VectorSubcoreMesh(
    core_axis_name="c", subcore_axis_name="s", num_cores=NC, num_subcores=NS
)


@functools.partial(
    pl.kernel,
    out_type=jax.ShapeDtypeStruct((NC * L,), jnp.float32),
    mesh=_mesh,
    compiler_params=pltpu.CompilerParams(needs_layout_passes=False),
    scratch_types=[
        pltpu.VMEM((2 * BLK,), jnp.float32),
        pltpu.VMEM((HBINS,), jnp.float32),
        pltpu.VMEM((SLICE,), jnp.float32),
        pltpu.VMEM((SLICE,), jnp.float32),
        pltpu.VMEM((L,), jnp.float32),
        pltpu.VMEM((NS * L,), jnp.float32),
        pltpu.VMEM_SHARED((NS * HBINS,), jnp.float32),
        pltpu.VMEM_SHARED((NS * L,), jnp.float32),
        pltpu.SemaphoreType.DMA,
        pltpu.SemaphoreType.DMA,
    ],
)
def _hist_loss(out_hbm, tgt_hbm, res_hbm, buf, hist, tsum, trow, accbuf,
               pbuf, hist_sh, part_sh, sem0, sem1):
    c = lax.axis_index("c")
    s = lax.axis_index("s")
    base = c * CORE_ELEMS + s * SUB_ELEMS

    zero16 = jnp.zeros((L,), jnp.float32)

    def _zero(i, carry):
        hist[pl.ds(i * L, L)] = zero16
        return carry

    lax.fori_loop(0, HBINS // L, _zero, 0)

    sems = (sem0, sem1)

    def _process(src_hbm, sign):
        val = jnp.full((L,), sign, jnp.float32)
        pltpu.make_async_copy(
            src_hbm.at[pl.ds(base, BLK)], buf.at[pl.ds(0, BLK)], sems[0]
        ).start()

        def _blk(g, carry):
            for k in range(2):
                b = g * 2 + k
                nxt = (k + 1) % 2

                @pl.when(b + 1 < NBLK)
                def _start_next():
                    pltpu.make_async_copy(
                        src_hbm.at[pl.ds(base + (b + 1) * BLK, BLK)],
                        buf.at[pl.ds(nxt * BLK, BLK)],
                        sems[nxt],
                    ).start()

                pltpu.make_async_copy(
                    src_hbm.at[pl.ds(base, BLK)], buf.at[pl.ds(k * BLK, BLK)],
                    sems[k]
                ).wait()

                goff = s * SUB_ELEMS + b * BLK
                cbase = (goff // CHAN_ELEMS) * NBINS

                def _vec(j, carry2):
                    for u in range(UNROLL):
                        v = buf[pl.ds(k * BLK + (j * UNROLL + u) * L, L)]
                        x = jnp.clip(v * float(NBINS), 0.0, float(NBINS - 1))
                        idx = x.astype(jnp.int32) + cbase
                        plsc.addupdate_scatter(hist, (idx,), val)
                    return carry2

                lax.fori_loop(0, VPB // UNROLL, _vec, 0)
            return carry

        lax.fori_loop(0, NBLK // 2, _blk, 0)

    _process(out_hbm, 1.0)
    _process(tgt_hbm, -1.0)

    pltpu.sync_copy(hist, hist_sh.at[pl.ds(s * HBINS, HBINS)])
    plsc.subcore_barrier()

    col = s * SLICE
    pltpu.sync_copy(hist_sh.at[pl.ds(col, SLICE)], tsum)

    def _merge_row(w, carry):
        pltpu.sync_copy(hist_sh.at[pl.ds(w * HBINS + col, SLICE)], trow)

        def _addvec(j, carry2):
            tsum[pl.ds(j * L, L)] = tsum[pl.ds(j * L, L)] + trow[pl.ds(j * L, L)]
            return carry2

        lax.fori_loop(0, SLICE // L, _addvec, 0)
        return carry

    lax.fori_loop(1, NS, _merge_row, 0)

    acc = zero16
    for j in range(SLICE // L):
        acc = acc + jnp.abs(tsum[pl.ds(j * L, L)])

    accbuf[...] = acc
    pltpu.sync_copy(accbuf, part_sh.at[pl.ds(s * L, L)])
    plsc.subcore_barrier()

    @pl.when(s == 0)
    def _finish():
        pltpu.sync_copy(part_sh, pbuf)
        tot = zero16
        for w in range(NS):
            tot = tot + pbuf[pl.ds(w * L, L)]
        accbuf[...] = tot * SCALE
        pltpu.sync_copy(accbuf, res_hbm.at[pl.ds(c * L, L)])


def kernel(output, target):
    parts = _hist_loss(output.reshape(-1), target.reshape(-1))
    return jnp.sum(parts)


# speedup vs baseline: 113.3834x; 3.0199x over previous
import functools

import jax
import jax.numpy as jnp
from jax import lax
from jax.experimental import pallas as pl
from jax.experimental.pallas import tpu as pltpu
from jax.experimental.pallas import tpu_sc as plsc

NBINS = 256
BATCH, CHANS, H, W = 8, 3, 512, 512
CHAN_ELEMS = H * W
N_CHAN = BATCH * CHANS
N_TOT = N_CHAN * CHAN_ELEMS
NC, NS, L = 2, 16, 16
CH_PER_CORE = N_CHAN // NC
CORE_ELEMS = CH_PER_CORE * CHAN_ELEMS
SUB_ELEMS = CORE_ELEMS // NS
BLK = 16384
NBLK = SUB_ELEMS // BLK
VPB = BLK // L
UNROLL = 8
HBINS = CH_PER_CORE * NBINS
SLICE = HBINS // NS
SCALE = 1.0 / float(N_CHAN * NBINS * CHAN_ELEMS)

_mesh = plsc.VectorSubcoreMesh(
    core_axis_name="c", subcore_axis_name="s", num_cores=NC, num_subcores=NS
)


@functools.partial(
    pl.kernel,
    out_type=jax.ShapeDtypeStruct((NC * L,), jnp.float32),
    mesh=_mesh,
    compiler_params=pltpu.CompilerParams(needs_layout_passes=False),
    scratch_types=[
        pltpu.VMEM((2 * BLK,), jnp.float32),
        pltpu.VMEM((HBINS,), jnp.float32),
        pltpu.VMEM((SLICE,), jnp.float32),
        pltpu.VMEM((SLICE,), jnp.float32),
        pltpu.VMEM((L,), jnp.float32),
        pltpu.VMEM((NS * L,), jnp.float32),
        pltpu.VMEM_SHARED((NS * HBINS,), jnp.float32),
        pltpu.VMEM_SHARED((NS * L,), jnp.float32),
        pltpu.SemaphoreType.DMA,
        pltpu.SemaphoreType.DMA,
    ],
)
def _hist_loss(out_hbm, tgt_hbm, res_hbm, buf, hist, tsum, trow, accbuf,
               pbuf, hist_sh, part_sh, sem0, sem1):
    c = lax.axis_index("c")
    s = lax.axis_index("s")
    base = c * CORE_ELEMS + s * SUB_ELEMS

    zero16 = jnp.zeros((L,), jnp.float32)

    def _zero(i, carry):
        hist[pl.ds(i * L, L)] = zero16
        return carry

    lax.fori_loop(0, HBINS // L, _zero, 0)

    sems = (sem0, sem1)

    def _process(src_hbm, sign):
        val = jnp.full((L,), sign, jnp.float32)
        pltpu.make_async_copy(
            src_hbm.at[pl.ds(base, BLK)], buf.at[pl.ds(0, BLK)], sems[0]
        ).start()

        def _blk(g, carry):
            for k in range(2):
                b = g * 2 + k
                nxt = (k + 1) % 2

                @pl.when(b + 1 < NBLK)
                def _start_next():
                    pltpu.make_async_copy(
                        src_hbm.at[pl.ds(base + (b + 1) * BLK, BLK)],
                        buf.at[pl.ds(nxt * BLK, BLK)],
                        sems[nxt],
                    ).start()

                pltpu.make_async_copy(
                    src_hbm.at[pl.ds(base, BLK)], buf.at[pl.ds(k * BLK, BLK)],
                    sems[k]
                ).wait()

                goff = s * SUB_ELEMS + b * BLK
                cbase = (goff // CHAN_ELEMS) * NBINS

                @plsc.parallel_loop(0, VPB, 1, unroll=UNROLL)
                def _vec(j):
                    v = buf[pl.ds(k * BLK + j * L, L)]
                    x = jnp.clip(v * float(NBINS), 0.0, float(NBINS - 1))
                    idx = x.astype(jnp.int32) + cbase
                    plsc.addupdate_scatter(hist, (idx,), val)
            return carry

        lax.fori_loop(0, NBLK // 2, _blk, 0)

    _process(out_hbm, 1.0)
    _process(tgt_hbm, -1.0)

    pltpu.sync_copy(hist, hist_sh.at[pl.ds(s * HBINS, HBINS)])
    plsc.subcore_barrier()

    col = s * SLICE
    pltpu.sync_copy(hist_sh.at[pl.ds(col, SLICE)], tsum)

    def _merge_row(w, carry):
        pltpu.sync_copy(hist_sh.at[pl.ds(w * HBINS + col, SLICE)], trow)

        def _addvec(j, carry2):
            tsum[pl.ds(j * L, L)] = tsum[pl.ds(j * L, L)] + trow[pl.ds(j * L, L)]
            return carry2

        lax.fori_loop(0, SLICE // L, _addvec, 0)
        return carry

    lax.fori_loop(1, NS, _merge_row, 0)

    acc = zero16
    for j in range(SLICE // L):
        acc = acc + jnp.abs(tsum[pl.ds(j * L, L)])

    accbuf[...] = acc
    pltpu.sync_copy(accbuf, part_sh.at[pl.ds(s * L, L)])
    plsc.subcore_barrier()

    @pl.when(s == 0)
    def _finish():
        pltpu.sync_copy(part_sh, pbuf)
        tot = zero16
        for w in range(NS):
            tot = tot + pbuf[pl.ds(w * L, L)]
        accbuf[...] = tot * SCALE
        pltpu.sync_copy(accbuf, res_hbm.at[pl.ds(c * L, L)])


def kernel(output, target):
    parts = _hist_loss(output.reshape(-1), target.reshape(-1))
    return jnp.sum(parts)


# speedup vs baseline: 178.9670x; 1.5784x over previous
import functools

import jax
import jax.numpy as jnp
from jax import lax
from jax.experimental import pallas as pl
from jax.experimental.pallas import tpu as pltpu
from jax.experimental.pallas import tpu_sc as plsc

NBINS = 256
BATCH, CHANS, H, W = 8, 3, 512, 512
CHAN_ELEMS = H * W
N_CHAN = BATCH * CHANS
N_TOT = N_CHAN * CHAN_ELEMS
NC, NS, L = 2, 16, 16
CH_PER_CORE = N_CHAN // NC
CORE_ELEMS = CH_PER_CORE * CHAN_ELEMS
SUB_ELEMS = CORE_ELEMS // NS
BLK = 16384
NBLK = SUB_ELEMS // BLK
VPB = BLK // L
UNROLL = 8
HBINS = CH_PER_CORE * NBINS
SLICE = HBINS // NS
SCALE = 1.0 / float(N_CHAN * NBINS * CHAN_ELEMS)

_mesh = plsc.VectorSubcoreMesh(
    core_axis_name="c", subcore_axis_name="s", num_cores=NC, num_subcores=NS
)


@functools.partial(
    pl.kernel,
    out_type=jax.ShapeDtypeStruct((NC * L,), jnp.float32),
    mesh=_mesh,
    compiler_params=pltpu.CompilerParams(needs_layout_passes=False),
    scratch_types=[
        pltpu.VMEM((2 * BLK,), jnp.float32),
        pltpu.VMEM((HBINS,), jnp.float32),
        pltpu.VMEM((SLICE,), jnp.float32),
        pltpu.VMEM((SLICE,), jnp.float32),
        pltpu.VMEM((L,), jnp.float32),
        pltpu.VMEM((NS * L,), jnp.float32),
        pltpu.VMEM_SHARED((NS * HBINS,), jnp.float32),
        pltpu.VMEM_SHARED((NS * L,), jnp.float32),
        pltpu.SemaphoreType.DMA,
        pltpu.SemaphoreType.DMA,
    ],
)
def _hist_loss(out_hbm, tgt_hbm, res_hbm, buf, hist, tsum, trow, accbuf,
               pbuf, hist_sh, part_sh, sem0, sem1):
    c = lax.axis_index("c")
    s = lax.axis_index("s")
    base = c * CORE_ELEMS + s * SUB_ELEMS

    zero16 = jnp.zeros((L,), jnp.float32)

    def _zero(i, carry):
        hist[pl.ds(i * L, L)] = zero16
        return carry

    lax.fori_loop(0, HBINS // L, _zero, 0)

    sems = (sem0, sem1)

    def _process(src_hbm, sign):
        val = jnp.full((L,), sign, jnp.float32)
        pltpu.make_async_copy(
            src_hbm.at[pl.ds(base, BLK)], buf.at[pl.ds(0, BLK)], sems[0]
        ).start()

        def _blk(g, carry):
            for k in range(2):
                b = g * 2 + k
                nxt = (k + 1) % 2

                @pl.when(b + 1 < NBLK)
                def _start_next():
                    pltpu.make_async_copy(
                        src_hbm.at[pl.ds(base + (b + 1) * BLK, BLK)],
                        buf.at[pl.ds(nxt * BLK, BLK)],
                        sems[nxt],
                    ).start()

                pltpu.make_async_copy(
                    src_hbm.at[pl.ds(base, BLK)], buf.at[pl.ds(k * BLK, BLK)],
                    sems[k]
                ).wait()

                goff = s * SUB_ELEMS + b * BLK
                cbase = (goff // CHAN_ELEMS) * NBINS

                @plsc.parallel_loop(0, VPB, 1, unroll=UNROLL)
                def _vec(j):
                    v = buf[pl.ds(k * BLK + j * L, L)]
                    x = jnp.clip(v * float(NBINS), 0.0, float(NBINS - 1))
                    idx = x.astype(jnp.int32) + cbase
                    plsc.addupdate_scatter(hist, (idx,), val)
            return carry

        lax.fori_loop(0, NBLK // 2, _blk, 0)

    _process(out_hbm, 1.0)
    _process(tgt_hbm, -1.0)

    pltpu.sync_copy(hist, hist_sh.at[pl.ds(s * HBINS, HBINS)])
    plsc.subcore_barrier()

    col = s * SLICE
    pltpu.sync_copy(hist_sh.at[pl.ds(col, SLICE)], tsum)

    def _merge_row(w, carry):
        pltpu.sync_copy(hist_sh.at[pl.ds(w * HBINS + col, SLICE)], trow)

        def _addvec(j, carry2):
            tsum[pl.ds(j * L, L)] = tsum[pl.ds(j * L, L)] + trow[pl.ds(j * L, L)]
            return carry2

        lax.fori_loop(0, SLICE // L, _addvec, 0)
        return carry

    lax.fori_loop(1, NS, _merge_row, 0)

    acc = zero16
    for j in range(SLICE // L):
        acc = acc + jnp.abs(tsum[pl.ds(j * L, L)])

    accbuf[...] = acc
    pltpu.sync_copy(accbuf, part_sh.at[pl.ds(s * L, L)])
    plsc.subcore_barrier()

    @pl.when(s == 0)
    def _finish():
        pltpu.sync_copy(part_sh, pbuf)
        tot = zero16
        for w in range(NS):
            tot = tot + pbuf[pl.ds(w * L, L)]
        accbuf[...] = tot * SCALE
        pltpu.sync_copy(accbuf, res_hbm.at[pl.ds(c * L, L)])


def _flat_tiled(x):
    x5 = x.reshape(N_CHAN, H // 8, 8, W // 128, 128)
    return x5.transpose(0, 1, 3, 2, 4).reshape(-1)


def kernel(output, target):
    parts = _hist_loss(_flat_tiled(output), _flat_tiled(target))
    return jnp.sum(parts)
